# Initial kernel scaffold; baseline (speedup 1.0000x reference)
import jax
import jax.numpy as jnp
from jax.experimental import pallas as pl


def kernel(feats, segment_ids, W, b):
    raise NotImplementedError("write your pallas kernel here")


# speedup vs baseline: 3.1190x; 3.1190x over previous
import jax
import jax.numpy as jnp
import numpy as np
from jax.experimental import pallas as pl
from jax.experimental.pallas import tpu as pltpu

N_NODES = 100000
D_FEAT = 128
N_TASKS = 4
N_GRAPHS = 2048

BLK = 1000


def _body(seg_ref, feats_ref, wt_ref, b_ref, w_out_ref, acc_ref):
    i = pl.program_id(0)

    @pl.when(i == 0)
    def _init():
        acc_ref[...] = jnp.zeros_like(acc_ref)

    feats = feats_ref[...]
    logits = jax.lax.dot_general(
        feats, wt_ref[...], (((1,), (0,)), ((), ())),
        preferred_element_type=jnp.float32)
    w = jax.nn.sigmoid(logits + b_ref[...])
    w_out_ref[...] = w

    weighted = (feats[:, None, :] * w[:, :, None]).reshape(BLK, N_TASKS * D_FEAT)

    seg = seg_ref[0, 0, :]
    gids = jax.lax.broadcasted_iota(jnp.int32, (BLK, N_GRAPHS), 1)
    onehot = (seg[:, None] == gids).astype(jnp.float32)
    part = jax.lax.dot_general(
        onehot, weighted, (((0,), (0,)), ((), ())),
        preferred_element_type=jnp.float32)
    acc_ref[...] += part


def kernel(feats, segment_ids, W, b):
    seg = segment_ids.astype(jnp.int32).reshape(N_NODES // BLK, 1, BLK)
    wt = W.T
    nblk = N_NODES // BLK

    w_all, acc = pl.pallas_call(
        _body,
        grid=(nblk,),
        in_specs=[
            pl.BlockSpec((1, 1, BLK), lambda i: (i, 0, 0)),
            pl.BlockSpec((BLK, D_FEAT), lambda i: (i, 0)),
            pl.BlockSpec((D_FEAT, N_TASKS), lambda i: (0, 0)),
            pl.BlockSpec((1, N_TASKS), lambda i: (0, 0)),
        ],
        out_specs=[
            pl.BlockSpec((BLK, N_TASKS), lambda i: (i, 0)),
            pl.BlockSpec((N_GRAPHS, N_TASKS * D_FEAT), lambda i: (0, 0)),
        ],
        out_shape=[
            jax.ShapeDtypeStruct((N_NODES, N_TASKS), jnp.float32),
            jax.ShapeDtypeStruct((N_GRAPHS, N_TASKS * D_FEAT), jnp.float32),
        ],
    )(seg, feats, wt, b.reshape(1, N_TASKS))

    readout = acc.reshape(N_GRAPHS, N_TASKS, D_FEAT).transpose(1, 0, 2)
    atoms = w_all.T.reshape(N_TASKS, N_NODES, 1)
    return (readout, atoms)
